# Initial kernel scaffold; baseline (speedup 1.0000x reference)
import jax
import jax.numpy as jnp
from jax.experimental import pallas as pl


def kernel(xyz1, xyz2, color1, color2, W0a, W0b, W0c, W1a, W1b, W1c, W2a, W2b, W2c, Wc1, Wc2, Wc3, Wd1, Wd2, Wf):
    raise NotImplementedError("write your pallas kernel here")


# speedup vs baseline: 2.7399x; 2.7399x over previous
import functools

import jax
import jax.numpy as jnp
from jax.experimental import pallas as pl
from jax.experimental.pallas import tpu as pltpu

_F32 = jnp.float32
_BF16 = jnp.bfloat16
_HI = jax.lax.Precision.HIGHEST


def _bdot(a, b):
    return jnp.dot(a.astype(_BF16), b.astype(_BF16),
                   preferred_element_type=_F32)


def _sq_dist(qx, rx):
    q2 = jnp.sum(qx * qx, axis=1, keepdims=True)
    r2 = jnp.transpose(jnp.sum(rx * rx, axis=1, keepdims=True))
    qr = jax.lax.dot_general(qx.astype(_BF16), rx.astype(_BF16),
                             (((1,), (1,)), ((), ())),
                             preferred_element_type=_F32)
    return q2 - 2.0 * qr + r2


def _sa_kernel(k, rcat_ref, qxp_ref, wa_ref, wb_ref, wc_ref, o_ref, d_scr):
    rcat = rcat_ref[0]
    qxp = qxp_ref[0]
    rx = rcat[:, :3]
    qx = qxp[:, :3]
    wa = wa_ref[...].astype(_BF16)
    wb = wb_ref[...].astype(_BF16)
    wc = wc_ref[...].astype(_BF16)

    d_scr[...] = _sq_dist(qx, rx)
    QB = qxp.shape[0]
    N = rcat.shape[0]
    iota = jax.lax.broadcasted_iota(jnp.int32, (QB, N), 1)
    acc0 = jnp.zeros((QB, wc.shape[1]), _F32)

    def body(_, acc):
        d = d_scr[...]
        m = jnp.min(d, axis=1, keepdims=True)
        j = jnp.min(jnp.where(d == m, iota, N), axis=1, keepdims=True)
        sel = iota == j
        d_scr[...] = jnp.where(sel, jnp.inf, d)
        eqf = jnp.where(sel, 1.0, 0.0)
        g = jnp.dot(eqf, rcat, precision=_HI,
                    preferred_element_type=_F32)
        h = (g - qxp).astype(_BF16)
        h = jnp.maximum(jnp.dot(h, wa, preferred_element_type=_F32), 0.0)
        h = jnp.maximum(jnp.dot(h.astype(_BF16), wb,
                                preferred_element_type=_F32), 0.0)
        h = jnp.maximum(jnp.dot(h.astype(_BF16), wc,
                                preferred_element_type=_F32), 0.0)
        return jnp.maximum(acc, h)

    o_ref[0] = jax.lax.fori_loop(0, k, body, acc0)


def _sa_level(rcat, qxp, Wa, Wb, Wc, k, qb):
    S, N, CI = rcat.shape
    M = qxp.shape[1]
    H3 = Wc.shape[1]
    return pl.pallas_call(
        functools.partial(_sa_kernel, k),
        grid=(S, M // qb),
        in_specs=[
            pl.BlockSpec((1, N, CI), lambda s, m: (s, 0, 0)),
            pl.BlockSpec((1, qb, CI), lambda s, m: (s, m, 0)),
            pl.BlockSpec(Wa.shape, lambda s, m: (0, 0)),
            pl.BlockSpec(Wb.shape, lambda s, m: (0, 0)),
            pl.BlockSpec(Wc.shape, lambda s, m: (0, 0)),
        ],
        out_specs=pl.BlockSpec((1, qb, H3), lambda s, m: (s, m, 0)),
        out_shape=jax.ShapeDtypeStruct((S, M, H3), _F32),
        scratch_shapes=[pltpu.VMEM((qb, N), _F32)],
    )(rcat, qxp, Wa, Wb, Wc)


def _cost_kernel(k, rcat_ref, x1_ref, f1_ref, x1p_ref, wc1_ref, wc2_ref,
                 wc3_ref, wd1_ref, wd2_ref, wf_ref, o_ref, d_scr):
    rcat = rcat_ref[0]
    x1 = x1_ref[0]
    f1 = f1_ref[0]
    x1p = x1p_ref[0]
    C = f1.shape[1]
    rx = rcat[:, C:]
    wc1 = wc1_ref[...].astype(_BF16)
    wc2 = wc2_ref[...].astype(_BF16)
    wc3 = wc3_ref[...].astype(_BF16)

    d_scr[...] = _sq_dist(x1, rx)
    M = x1.shape[0]
    N = rcat.shape[0]
    iota = jax.lax.broadcasted_iota(jnp.int32, (M, N), 1)
    acc0 = jnp.zeros((M, wc3.shape[1]), _F32)

    def body(_, acc):
        d = d_scr[...]
        m = jnp.min(d, axis=1, keepdims=True)
        j = jnp.min(jnp.where(d == m, iota, N), axis=1, keepdims=True)
        sel = iota == j
        d_scr[...] = jnp.where(sel, jnp.inf, d)
        eqf = jnp.where(sel, 1.0, 0.0)
        g = jnp.dot(eqf, rcat, precision=_HI,
                    preferred_element_type=_F32)
        h = (jnp.concatenate([f1, g], axis=1) - x1p).astype(_BF16)
        h = jnp.maximum(jnp.dot(h, wc1, preferred_element_type=_F32), 0.0)
        h = jnp.maximum(jnp.dot(h.astype(_BF16), wc2,
                                preferred_element_type=_F32), 0.0)
        h = jnp.maximum(jnp.dot(h.astype(_BF16), wc3,
                                preferred_element_type=_F32), 0.0)
        return jnp.maximum(acc, h)

    c = jax.lax.fori_loop(0, k, body, acc0)
    h2 = jnp.concatenate([c, f1], axis=1).astype(_BF16)
    h2 = jnp.maximum(jnp.dot(h2, wd1_ref[...].astype(_BF16),
                             preferred_element_type=_F32), 0.0)
    h2 = jnp.maximum(jnp.dot(h2.astype(_BF16), wd2_ref[...].astype(_BF16),
                             preferred_element_type=_F32), 0.0)
    o_ref[0] = jnp.dot(h2.astype(_BF16), wf_ref[...].astype(_BF16),
                       preferred_element_type=_F32)


def _cost_head(x1, f1, x2, f2, Wc1, Wc2, Wc3, Wd1, Wd2, Wf, k):
    B, M, _ = x1.shape
    N = x2.shape[1]
    C = f1.shape[2]
    rcat = jnp.concatenate([f2, x2], axis=2)
    x1p = jnp.concatenate([jnp.zeros((B, M, 2 * C), _F32), x1], axis=2)
    full = lambda w: pl.BlockSpec(w.shape, lambda b: tuple(0 for _ in w.shape))
    return pl.pallas_call(
        functools.partial(_cost_kernel, k),
        grid=(B,),
        in_specs=[
            pl.BlockSpec((1, N, C + 3), lambda b: (b, 0, 0)),
            pl.BlockSpec((1, M, 3), lambda b: (b, 0, 0)),
            pl.BlockSpec((1, M, C), lambda b: (b, 0, 0)),
            pl.BlockSpec((1, M, 2 * C + 3), lambda b: (b, 0, 0)),
            full(Wc1), full(Wc2), full(Wc3), full(Wd1), full(Wd2), full(Wf),
        ],
        out_specs=pl.BlockSpec((1, M, Wf.shape[1]), lambda b: (b, 0, 0)),
        out_shape=jax.ShapeDtypeStruct((B, M, Wf.shape[1]), _F32),
        scratch_shapes=[pltpu.VMEM((M, N), _F32)],
    )(rcat, x1, f1, x1p, Wc1, Wc2, Wc3, Wd1, Wd2, Wf)


def _pad_q(qx, c):
    S, M, _ = qx.shape
    return jnp.concatenate([qx, jnp.zeros((S, M, c), _F32)], axis=2)


def kernel(xyz1, xyz2, color1, color2, W0a, W0b, W0c, W1a, W1b, W1c,
           W2a, W2b, W2c, Wc1, Wc2, Wc3, Wd1, Wd2, Wf):
    xyz1 = jnp.transpose(xyz1, (0, 2, 1))
    xyz2 = jnp.transpose(xyz2, (0, 2, 1))
    center = jnp.mean(xyz1, axis=1, keepdims=True)
    xyz = jnp.concatenate([xyz1 - center, xyz2 - center], 0)
    col = jnp.transpose(jnp.concatenate([color1, color2], 0), (0, 2, 1))

    q0 = xyz[:, ::4]
    f0 = _sa_level(jnp.concatenate([xyz, col], 2), _pad_q(q0, 3),
                   W0a, W0b, W0c, k=32, qb=256)
    q1 = q0[:, ::2]
    f1 = _sa_level(jnp.concatenate([q0, f0], 2), _pad_q(q1, 32),
                   W1a, W1b, W1c, k=24, qb=256)
    q2 = q1[:, ::4]
    f2 = _sa_level(jnp.concatenate([q1, f1], 2), _pad_q(q2, 64),
                   W2a, W2b, W2c, k=16, qb=256)

    B = xyz1.shape[0]
    return _cost_head(q2[:B], f2[:B], q2[B:], f2[B:],
                      Wc1, Wc2, Wc3, Wd1, Wd2, Wf, k=32)


# speedup vs baseline: 5.0958x; 1.8598x over previous
import functools

import jax
import jax.numpy as jnp
from jax.experimental import pallas as pl
from jax.experimental.pallas import tpu as pltpu

_F32 = jnp.float32
_BF16 = jnp.bfloat16
_HI = jax.lax.Precision.HIGHEST
_CH = 128


def _bf16(x):
    return x.astype(_BF16)


def _sq_dist(qx, rx):
    q2 = jnp.sum(qx * qx, axis=1, keepdims=True)
    r2 = jnp.transpose(jnp.sum(rx * rx, axis=1, keepdims=True))
    qr = jax.lax.dot_general(_bf16(qx), _bf16(rx),
                             (((1,), (1,)), ((), ())),
                             preferred_element_type=_F32)
    return q2 - 2.0 * qr + r2


def _mlp_max(g, qxp, wa, wb, wc, acc):
    h = _bf16(g - qxp)
    h = jnp.maximum(jnp.dot(h, wa, preferred_element_type=_F32), 0.0)
    h = jnp.maximum(jnp.dot(_bf16(h), wb, preferred_element_type=_F32), 0.0)
    h = jnp.maximum(jnp.dot(_bf16(h), wc, preferred_element_type=_F32), 0.0)
    return jnp.maximum(acc, h)


def _sa_kernel_chunked(k, rx_ref, rs_ref, sm_ref, qxp_ref, wa_ref, wb_ref,
                       wc_ref, o_ref, d_scr):
    rx = rx_ref[0]
    rs = rs_ref[0]
    sm = sm_ref[...]
    qxp = qxp_ref[0]
    qx = qxp[:, :3]
    wa = _bf16(wa_ref[...])
    wb = _bf16(wb_ref[...])
    wc = _bf16(wc_ref[...])

    d0 = _sq_dist(qx, rx)
    d_scr[...] = d0
    QB = qxp.shape[0]
    N = rx.shape[0]
    NC = rs.shape[0]
    W = qxp.shape[1]
    iota = jax.lax.broadcasted_iota(jnp.int32, (QB, N), 1)
    iota_c = jax.lax.broadcasted_iota(jnp.int32, (QB, NC), 1)
    iota_l = jax.lax.broadcasted_iota(jnp.int32, (QB, _CH), 1)
    m0 = jnp.min(d0, axis=1, keepdims=True)
    acc0 = jnp.zeros((QB, wc.shape[1]), _F32)

    def body(_, carry):
        m, acc = carry
        d = d_scr[...]
        j = jnp.min(jnp.where(d == m, iota, N), axis=1, keepdims=True)
        masked = jnp.where(iota == j, jnp.inf, d)
        d_scr[...] = masked
        m_next = jnp.min(masked, axis=1, keepdims=True)
        oc = jnp.where(iota_c == (j >> 7), 1.0, 0.0)
        os = jnp.where(iota_l == (j & 127), 1.0, 0.0)
        t1 = jnp.dot(oc, rs, precision=_HI,
                     preferred_element_type=_F32)
        p = t1 * jnp.concatenate([os] * W, axis=1)
        g = jnp.dot(p, sm, precision=_HI,
                    preferred_element_type=_F32)
        return m_next, _mlp_max(g, qxp, wa, wb, wc, acc)

    _, acc = jax.lax.fori_loop(0, k, body, (m0, acc0))
    o_ref[0] = acc


def _sa_kernel(k, rcat_ref, qxp_ref, wa_ref, wb_ref, wc_ref, o_ref, d_scr):
    rcat = rcat_ref[0]
    qxp = qxp_ref[0]
    rx = rcat[:, :3]
    qx = qxp[:, :3]
    wa = _bf16(wa_ref[...])
    wb = _bf16(wb_ref[...])
    wc = _bf16(wc_ref[...])

    d0 = _sq_dist(qx, rx)
    d_scr[...] = d0
    QB = qxp.shape[0]
    N = rcat.shape[0]
    iota = jax.lax.broadcasted_iota(jnp.int32, (QB, N), 1)
    m0 = jnp.min(d0, axis=1, keepdims=True)
    acc0 = jnp.zeros((QB, wc.shape[1]), _F32)

    def body(_, carry):
        m, acc = carry
        d = d_scr[...]
        j = jnp.min(jnp.where(d == m, iota, N), axis=1, keepdims=True)
        sel = iota == j
        masked = jnp.where(sel, jnp.inf, d)
        d_scr[...] = masked
        m_next = jnp.min(masked, axis=1, keepdims=True)
        eqf = jnp.where(sel, 1.0, 0.0)
        g = jnp.dot(eqf, rcat, precision=_HI,
                    preferred_element_type=_F32)
        return m_next, _mlp_max(g, qxp, wa, wb, wc, acc)

    _, acc = jax.lax.fori_loop(0, k, body, (m0, acc0))
    o_ref[0] = acc


def _sa_level(xyz_r, feat, qxp, Wa, Wb, Wc, k, qb):
    S, N, _ = xyz_r.shape
    M = qxp.shape[1]
    W = qxp.shape[2]
    H3 = Wc.shape[1]
    grid = (S, M // qb)
    out_shape = jax.ShapeDtypeStruct((S, M, H3), _F32)
    out_spec = pl.BlockSpec((1, qb, H3), lambda s, m: (s, m, 0))
    wspecs = [pl.BlockSpec(Wa.shape, lambda s, m: (0, 0)),
              pl.BlockSpec(Wb.shape, lambda s, m: (0, 0)),
              pl.BlockSpec(Wc.shape, lambda s, m: (0, 0))]
    scratch = [pltpu.VMEM((qb, N), _F32)]
    qspec = pl.BlockSpec((1, qb, W), lambda s, m: (s, m, 0))
    if N >= 4096:
        NC = N // _CH
        rcat = jnp.concatenate([xyz_r, feat], axis=2)
        rs = rcat.reshape(S, NC, _CH, W).transpose(0, 1, 3, 2)
        rs = rs.reshape(S, NC, W * _CH)
        sm = (jnp.arange(W * _CH)[:, None] // _CH
              == jnp.arange(W)[None, :]).astype(_F32)
        return pl.pallas_call(
            functools.partial(_sa_kernel_chunked, k),
            grid=grid,
            in_specs=[
                pl.BlockSpec((1, N, 3), lambda s, m: (s, 0, 0)),
                pl.BlockSpec((1, NC, W * _CH), lambda s, m: (s, 0, 0)),
                pl.BlockSpec(sm.shape, lambda s, m: (0, 0)),
                qspec,
            ] + wspecs,
            out_specs=out_spec,
            out_shape=out_shape,
            scratch_shapes=scratch,
        )(xyz_r, rs, sm, qxp, Wa, Wb, Wc)
    rcat = jnp.concatenate([xyz_r, feat], axis=2)
    return pl.pallas_call(
        functools.partial(_sa_kernel, k),
        grid=grid,
        in_specs=[
            pl.BlockSpec((1, N, W), lambda s, m: (s, 0, 0)),
            qspec,
        ] + wspecs,
        out_specs=out_spec,
        out_shape=out_shape,
        scratch_shapes=scratch,
    )(rcat, qxp, Wa, Wb, Wc)


def _cost_kernel(k, rcat_ref, x1_ref, f1_ref, x1p_ref, wc1_ref, wc2_ref,
                 wc3_ref, wd1_ref, wd2_ref, wf_ref, o_ref, d_scr):
    rcat = rcat_ref[0]
    x1 = x1_ref[0]
    f1 = f1_ref[0]
    x1p = x1p_ref[0]
    C = f1.shape[1]
    rx = rcat[:, C:]
    wc1 = _bf16(wc1_ref[...])
    wc2 = _bf16(wc2_ref[...])
    wc3 = _bf16(wc3_ref[...])

    d0 = _sq_dist(x1, rx)
    d_scr[...] = d0
    M = x1.shape[0]
    N = rcat.shape[0]
    iota = jax.lax.broadcasted_iota(jnp.int32, (M, N), 1)
    m0 = jnp.min(d0, axis=1, keepdims=True)
    acc0 = jnp.zeros((M, wc3.shape[1]), _F32)

    def body(_, carry):
        m, acc = carry
        d = d_scr[...]
        j = jnp.min(jnp.where(d == m, iota, N), axis=1, keepdims=True)
        sel = iota == j
        masked = jnp.where(sel, jnp.inf, d)
        d_scr[...] = masked
        m_next = jnp.min(masked, axis=1, keepdims=True)
        eqf = jnp.where(sel, 1.0, 0.0)
        g = jnp.dot(eqf, rcat, precision=_HI,
                    preferred_element_type=_F32)
        h = _bf16(jnp.concatenate([f1, g], axis=1) - x1p)
        h = jnp.maximum(jnp.dot(h, wc1, preferred_element_type=_F32), 0.0)
        h = jnp.maximum(jnp.dot(_bf16(h), wc2,
                                preferred_element_type=_F32), 0.0)
        h = jnp.maximum(jnp.dot(_bf16(h), wc3,
                                preferred_element_type=_F32), 0.0)
        return m_next, jnp.maximum(acc, h)

    _, c = jax.lax.fori_loop(0, k, body, (m0, acc0))
    h2 = _bf16(jnp.concatenate([c, f1], axis=1))
    h2 = jnp.maximum(jnp.dot(h2, _bf16(wd1_ref[...]),
                             preferred_element_type=_F32), 0.0)
    h2 = jnp.maximum(jnp.dot(_bf16(h2), _bf16(wd2_ref[...]),
                             preferred_element_type=_F32), 0.0)
    o_ref[0] = jnp.dot(_bf16(h2), _bf16(wf_ref[...]),
                       preferred_element_type=_F32)


def _cost_head(x1, f1, x2, f2, Wc1, Wc2, Wc3, Wd1, Wd2, Wf, k):
    B, M, _ = x1.shape
    N = x2.shape[1]
    C = f1.shape[2]
    rcat = jnp.concatenate([f2, x2], axis=2)
    x1p = jnp.concatenate([jnp.zeros((B, M, 2 * C), _F32), x1], axis=2)
    full = lambda w: pl.BlockSpec(w.shape, lambda b: tuple(0 for _ in w.shape))
    return pl.pallas_call(
        functools.partial(_cost_kernel, k),
        grid=(B,),
        in_specs=[
            pl.BlockSpec((1, N, C + 3), lambda b: (b, 0, 0)),
            pl.BlockSpec((1, M, 3), lambda b: (b, 0, 0)),
            pl.BlockSpec((1, M, C), lambda b: (b, 0, 0)),
            pl.BlockSpec((1, M, 2 * C + 3), lambda b: (b, 0, 0)),
            full(Wc1), full(Wc2), full(Wc3), full(Wd1), full(Wd2), full(Wf),
        ],
        out_specs=pl.BlockSpec((1, M, Wf.shape[1]), lambda b: (b, 0, 0)),
        out_shape=jax.ShapeDtypeStruct((B, M, Wf.shape[1]), _F32),
        scratch_shapes=[pltpu.VMEM((M, N), _F32)],
    )(rcat, x1, f1, x1p, Wc1, Wc2, Wc3, Wd1, Wd2, Wf)


def _pad_q(qx, c):
    S, M, _ = qx.shape
    return jnp.concatenate([qx, jnp.zeros((S, M, c), _F32)], axis=2)


def kernel(xyz1, xyz2, color1, color2, W0a, W0b, W0c, W1a, W1b, W1c,
           W2a, W2b, W2c, Wc1, Wc2, Wc3, Wd1, Wd2, Wf):
    xyz1 = jnp.transpose(xyz1, (0, 2, 1))
    xyz2 = jnp.transpose(xyz2, (0, 2, 1))
    center = jnp.mean(xyz1, axis=1, keepdims=True)
    xyz = jnp.concatenate([xyz1 - center, xyz2 - center], 0)
    col = jnp.transpose(jnp.concatenate([color1, color2], 0), (0, 2, 1))

    q0 = xyz[:, ::4]
    f0 = _sa_level(xyz, col, _pad_q(q0, 3), W0a, W0b, W0c, k=32, qb=256)
    q1 = q0[:, ::2]
    f1 = _sa_level(q0, f0, _pad_q(q1, 32), W1a, W1b, W1c, k=24, qb=256)
    q2 = q1[:, ::4]
    f2 = _sa_level(q1, f1, _pad_q(q2, 64), W2a, W2b, W2c, k=16, qb=256)

    B = xyz1.shape[0]
    return _cost_head(q2[:B], f2[:B], q2[B:], f2[B:],
                      Wc1, Wc2, Wc3, Wd1, Wd2, Wf, k=32)


# speedup vs baseline: 7.9473x; 1.5596x over previous
import functools

import jax
import jax.numpy as jnp
from jax.experimental import pallas as pl
from jax.experimental.pallas import tpu as pltpu

_F32 = jnp.float32
_BF16 = jnp.bfloat16
_HI = jax.lax.Precision.HIGHEST
_CH = 128


def _bf16(x):
    return x.astype(_BF16)


def _hilo(x):
    hi = x.astype(_BF16)
    return hi, (x - hi.astype(_F32)).astype(_BF16)


def _sq_dist(qx, rx, rxt):
    q2 = jnp.sum(qx * qx, axis=1, keepdims=True)
    r2 = jnp.sum(rxt * rxt, axis=0, keepdims=True)
    qr = jax.lax.dot_general(_bf16(qx), _bf16(rx),
                             (((1,), (1,)), ((), ())),
                             preferred_element_type=_F32)
    return q2 - 2.0 * qr + r2


def _mlp_max(g, qxp, wa, wb, wc, acc):
    h = _bf16(g - qxp)
    h = jnp.maximum(jnp.dot(h, wa, preferred_element_type=_F32), 0.0)
    h = jnp.maximum(jnp.dot(_bf16(h), wb, preferred_element_type=_F32), 0.0)
    h = jnp.maximum(jnp.dot(_bf16(h), wc, preferred_element_type=_F32), 0.0)
    return jnp.maximum(acc, h)


def _sa_kernel_chunked(k, rx_ref, rxt_ref, rsh_ref, rsl_ref, qxp_ref, wa_ref,
                       wb_ref, wc_ref, o_ref, d_scr):
    rx = rx_ref[0]
    rxt = rxt_ref[0]
    rsh = rsh_ref[0]
    rsl = rsl_ref[0]
    qxp = qxp_ref[0]
    qx = qxp[:, :3]
    wa = _bf16(wa_ref[...])
    wb = _bf16(wb_ref[...])
    wc = _bf16(wc_ref[...])

    d0 = _sq_dist(qx, rx, rxt)
    d_scr[...] = d0
    QB = qxp.shape[0]
    N = rx.shape[0]
    NC = rsh.shape[0]
    W = qxp.shape[1]
    iota = jax.lax.broadcasted_iota(jnp.int32, (QB, N), 1)
    iota_c = jax.lax.broadcasted_iota(jnp.int32, (QB, NC), 1)
    iota_l = jax.lax.broadcasted_iota(jnp.int32, (QB, _CH), 1)
    m0 = jnp.min(d0, axis=1, keepdims=True)
    acc0 = jnp.zeros((QB, wc.shape[1]), _F32)

    def body(_, carry):
        m, acc = carry
        d = d_scr[...]
        j = jnp.min(jnp.where(d == m, iota, N), axis=1, keepdims=True)
        masked = jnp.where(iota == j, jnp.inf, d)
        d_scr[...] = masked
        m_next = jnp.min(masked, axis=1, keepdims=True)
        oc = _bf16(jnp.where(iota_c == (j >> 7), 1.0, 0.0))
        os = jnp.where(iota_l == (j & 127), 1.0, 0.0)
        t1 = (jnp.dot(oc, rsh, preferred_element_type=_F32)
              + jnp.dot(oc, rsl, preferred_element_type=_F32))
        g = jnp.concatenate(
            [jnp.sum(t1[:, w * _CH:(w + 1) * _CH] * os, axis=1,
                     keepdims=True) for w in range(W)], axis=1)
        return m_next, _mlp_max(g, qxp, wa, wb, wc, acc)

    _, acc = jax.lax.fori_loop(0, k, body, (m0, acc0))
    o_ref[0] = acc


def _sa_kernel(k, rx_ref, rxt_ref, rch_ref, rcl_ref, qxp_ref, wa_ref, wb_ref,
               wc_ref, o_ref, d_scr):
    rx = rx_ref[0]
    rxt = rxt_ref[0]
    rch = rch_ref[0]
    rcl = rcl_ref[0]
    qxp = qxp_ref[0]
    qx = qxp[:, :3]
    wa = _bf16(wa_ref[...])
    wb = _bf16(wb_ref[...])
    wc = _bf16(wc_ref[...])

    d0 = _sq_dist(qx, rx, rxt)
    d_scr[...] = d0
    QB = qxp.shape[0]
    N = rx.shape[0]
    iota = jax.lax.broadcasted_iota(jnp.int32, (QB, N), 1)
    m0 = jnp.min(d0, axis=1, keepdims=True)
    acc0 = jnp.zeros((QB, wc.shape[1]), _F32)

    def body(_, carry):
        m, acc = carry
        d = d_scr[...]
        j = jnp.min(jnp.where(d == m, iota, N), axis=1, keepdims=True)
        sel = iota == j
        masked = jnp.where(sel, jnp.inf, d)
        d_scr[...] = masked
        m_next = jnp.min(masked, axis=1, keepdims=True)
        eqf = _bf16(jnp.where(sel, 1.0, 0.0))
        g = (jnp.dot(eqf, rch, preferred_element_type=_F32)
             + jnp.dot(eqf, rcl, preferred_element_type=_F32))
        return m_next, _mlp_max(g, qxp, wa, wb, wc, acc)

    _, acc = jax.lax.fori_loop(0, k, body, (m0, acc0))
    o_ref[0] = acc


def _sa_level(xyz_r, xyz_t, feat, qxp, Wa, Wb, Wc, k, qb):
    S, N, _ = xyz_r.shape
    M = qxp.shape[1]
    W = qxp.shape[2]
    H3 = Wc.shape[1]
    grid = (S, M // qb)
    out_shape = jax.ShapeDtypeStruct((S, M, H3), _F32)
    out_spec = pl.BlockSpec((1, qb, H3), lambda s, m: (s, m, 0))
    wspecs = [pl.BlockSpec(Wa.shape, lambda s, m: (0, 0)),
              pl.BlockSpec(Wb.shape, lambda s, m: (0, 0)),
              pl.BlockSpec(Wc.shape, lambda s, m: (0, 0))]
    scratch = [pltpu.VMEM((qb, N), _F32)]
    qspec = pl.BlockSpec((1, qb, W), lambda s, m: (s, m, 0))
    xspecs = [pl.BlockSpec((1, N, 3), lambda s, m: (s, 0, 0)),
              pl.BlockSpec((1, 3, N), lambda s, m: (s, 0, 0))]
    rcat = jnp.concatenate([xyz_r, feat], axis=2)
    if N >= 4096:
        NC = N // _CH
        rs = rcat.reshape(S, NC, _CH, W).transpose(0, 1, 3, 2)
        rsh, rsl = _hilo(rs.reshape(S, NC, W * _CH))
        return pl.pallas_call(
            functools.partial(_sa_kernel_chunked, k),
            grid=grid,
            in_specs=xspecs + [
                pl.BlockSpec((1, NC, W * _CH), lambda s, m: (s, 0, 0)),
                pl.BlockSpec((1, NC, W * _CH), lambda s, m: (s, 0, 0)),
                qspec,
            ] + wspecs,
            out_specs=out_spec,
            out_shape=out_shape,
            scratch_shapes=scratch,
        )(xyz_r, xyz_t, rsh, rsl, qxp, Wa, Wb, Wc)
    rch, rcl = _hilo(rcat)
    return pl.pallas_call(
        functools.partial(_sa_kernel, k),
        grid=grid,
        in_specs=xspecs + [
            pl.BlockSpec((1, N, W), lambda s, m: (s, 0, 0)),
            pl.BlockSpec((1, N, W), lambda s, m: (s, 0, 0)),
            qspec,
        ] + wspecs,
        out_specs=out_spec,
        out_shape=out_shape,
        scratch_shapes=scratch,
    )(xyz_r, xyz_t, rch, rcl, qxp, Wa, Wb, Wc)


def _cost_kernel(k, rcat_ref, x1_ref, f1_ref, x1p_ref, wc1_ref, wc2_ref,
                 wc3_ref, wd1_ref, wd2_ref, wf_ref, o_ref, d_scr):
    rcat = rcat_ref[0]
    x1 = x1_ref[0]
    f1 = f1_ref[0]
    x1p = x1p_ref[0]
    C = f1.shape[1]
    rx = rcat[:, C:]
    wc1 = _bf16(wc1_ref[...])
    wc2 = _bf16(wc2_ref[...])
    wc3 = _bf16(wc3_ref[...])

    q2 = jnp.sum(x1 * x1, axis=1, keepdims=True)
    r2 = jnp.transpose(jnp.sum(rx * rx, axis=1, keepdims=True))
    qr = jax.lax.dot_general(_bf16(x1), _bf16(rx),
                             (((1,), (1,)), ((), ())),
                             preferred_element_type=_F32)
    d0 = q2 - 2.0 * qr + r2
    d_scr[...] = d0
    M = x1.shape[0]
    N = rcat.shape[0]
    iota = jax.lax.broadcasted_iota(jnp.int32, (M, N), 1)
    m0 = jnp.min(d0, axis=1, keepdims=True)
    acc0 = jnp.zeros((M, wc3.shape[1]), _F32)

    def body(_, carry):
        m, acc = carry
        d = d_scr[...]
        j = jnp.min(jnp.where(d == m, iota, N), axis=1, keepdims=True)
        sel = iota == j
        masked = jnp.where(sel, jnp.inf, d)
        d_scr[...] = masked
        m_next = jnp.min(masked, axis=1, keepdims=True)
        eqf = jnp.where(sel, 1.0, 0.0)
        g = jnp.dot(eqf, rcat, precision=_HI,
                    preferred_element_type=_F32)
        h = _bf16(jnp.concatenate([f1, g], axis=1) - x1p)
        h = jnp.maximum(jnp.dot(h, wc1, preferred_element_type=_F32), 0.0)
        h = jnp.maximum(jnp.dot(_bf16(h), wc2,
                                preferred_element_type=_F32), 0.0)
        h = jnp.maximum(jnp.dot(_bf16(h), wc3,
                                preferred_element_type=_F32), 0.0)
        return m_next, jnp.maximum(acc, h)

    _, c = jax.lax.fori_loop(0, k, body, (m0, acc0))
    h2 = _bf16(jnp.concatenate([c, f1], axis=1))
    h2 = jnp.maximum(jnp.dot(h2, _bf16(wd1_ref[...]),
                             preferred_element_type=_F32), 0.0)
    h2 = jnp.maximum(jnp.dot(_bf16(h2), _bf16(wd2_ref[...]),
                             preferred_element_type=_F32), 0.0)
    o_ref[0] = jnp.dot(_bf16(h2), _bf16(wf_ref[...]),
                       preferred_element_type=_F32)


def _cost_head(x1, f1, x2, f2, Wc1, Wc2, Wc3, Wd1, Wd2, Wf, k):
    B, M, _ = x1.shape
    N = x2.shape[1]
    C = f1.shape[2]
    rcat = jnp.concatenate([f2, x2], axis=2)
    x1p = jnp.concatenate([jnp.zeros((B, M, 2 * C), _F32), x1], axis=2)
    full = lambda w: pl.BlockSpec(w.shape, lambda b: tuple(0 for _ in w.shape))
    return pl.pallas_call(
        functools.partial(_cost_kernel, k),
        grid=(B,),
        in_specs=[
            pl.BlockSpec((1, N, C + 3), lambda b: (b, 0, 0)),
            pl.BlockSpec((1, M, 3), lambda b: (b, 0, 0)),
            pl.BlockSpec((1, M, C), lambda b: (b, 0, 0)),
            pl.BlockSpec((1, M, 2 * C + 3), lambda b: (b, 0, 0)),
            full(Wc1), full(Wc2), full(Wc3), full(Wd1), full(Wd2), full(Wf),
        ],
        out_specs=pl.BlockSpec((1, M, Wf.shape[1]), lambda b: (b, 0, 0)),
        out_shape=jax.ShapeDtypeStruct((B, M, Wf.shape[1]), _F32),
        scratch_shapes=[pltpu.VMEM((M, N), _F32)],
    )(rcat, x1, f1, x1p, Wc1, Wc2, Wc3, Wd1, Wd2, Wf)


def _pad_q(qx, c):
    S, M, _ = qx.shape
    return jnp.concatenate([qx, jnp.zeros((S, M, c), _F32)], axis=2)


def kernel(xyz1, xyz2, color1, color2, W0a, W0b, W0c, W1a, W1b, W1c,
           W2a, W2b, W2c, Wc1, Wc2, Wc3, Wd1, Wd2, Wf):
    xyz1 = jnp.transpose(xyz1, (0, 2, 1))
    xyz2 = jnp.transpose(xyz2, (0, 2, 1))
    center = jnp.mean(xyz1, axis=1, keepdims=True)
    xyz = jnp.concatenate([xyz1 - center, xyz2 - center], 0)
    col = jnp.transpose(jnp.concatenate([color1, color2], 0), (0, 2, 1))
    xyzt = jnp.transpose(xyz, (0, 2, 1))

    q0 = xyz[:, ::4]
    f0 = _sa_level(xyz, xyzt, col, _pad_q(q0, 3), W0a, W0b, W0c, k=32, qb=256)
    q1 = q0[:, ::2]
    f1 = _sa_level(q0, jnp.transpose(q0, (0, 2, 1)), f0, _pad_q(q1, 32),
                   W1a, W1b, W1c, k=24, qb=256)
    q2 = q1[:, ::4]
    f2 = _sa_level(q1, jnp.transpose(q1, (0, 2, 1)), f1, _pad_q(q2, 64),
                   W2a, W2b, W2c, k=16, qb=256)

    B = xyz1.shape[0]
    return _cost_head(q2[:B], f2[:B], q2[B:], f2[B:],
                      Wc1, Wc2, Wc3, Wd1, Wd2, Wf, k=32)
